# baseline (speedup 1.0000x reference)
import functools

import jax
import jax.numpy as jnp
from jax.experimental import pallas as pl
from jax.experimental.pallas import tpu as pltpu

_NE = 8
_LANES = 128
_TILE = 512


def _moe_kernel(x_ref, rw_ref, w1_ref, b1_ref, w2_ref, b2_ref,
                out_ref, loss_ref, acc_ref, *, tokens, grid):
    step = pl.program_id(0)

    @pl.when(step == 0)
    def _init():
        acc_ref[...] = jnp.zeros_like(acc_ref)

    x = x_ref[...]

    logits = jnp.dot(x, rw_ref[...], preferred_element_type=jnp.float32)
    lane = jax.lax.broadcasted_iota(jnp.int32, logits.shape, 1)
    m = jnp.max(logits, axis=1, keepdims=True)
    e = jnp.exp(logits - m)
    probs = e / jnp.sum(e, axis=1, keepdims=True)

    v1 = jnp.max(probs, axis=1, keepdims=True)
    i1 = jnp.min(jnp.where(probs == v1, lane, _LANES), axis=1, keepdims=True)
    mask1 = lane == i1
    probs_rest = jnp.where(mask1, -1.0, probs)
    v2 = jnp.max(probs_rest, axis=1, keepdims=True)
    i2 = jnp.min(jnp.where(probs_rest == v2, lane, _LANES), axis=1, keepdims=True)
    mask2 = lane == i2

    s = v1 + v2
    w = v1 / s + v2 / s

    cnt = jnp.sum((mask1 | mask2).astype(jnp.float32), axis=0, keepdims=True)
    psum = jnp.sum(probs, axis=0, keepdims=True)
    acc_ref[0:1, 0:_NE] += cnt
    acc_ref[1:2, 0:_NE] += psum

    h = jnp.dot(x, w1_ref[...], preferred_element_type=jnp.float32) + b1_ref[...]
    a = jax.nn.gelu(h)
    y = jnp.dot(a, w2_ref[...], preferred_element_type=jnp.float32) + b2_ref[...]
    out_ref[...] = y * w

    @pl.when(step == grid - 1)
    def _finish():
        inv_t = 1.0 / tokens
        density = acc_ref[0:1, 0:_NE] * inv_t
        proxy = acc_ref[1:2, 0:_NE] * inv_t
        loss_ref[0, 0] = jnp.sum(density * proxy) * float(_NE)


def kernel(x, router_weights, W1, b1, W2, b2):
    B, S, H = x.shape
    F = W1.shape[1]
    T = B * S
    xs = x.reshape(T, H)
    b1r = b1.reshape(1, F)
    b2r = b2.reshape(1, H)
    grid = T // _TILE

    out, loss = pl.pallas_call(
        functools.partial(_moe_kernel, tokens=float(T), grid=grid),
        grid=(grid,),
        in_specs=[
            pl.BlockSpec((_TILE, H), lambda i: (i, 0)),
            pl.BlockSpec((H, _NE), lambda i: (0, 0)),
            pl.BlockSpec((H, F), lambda i: (0, 0)),
            pl.BlockSpec((1, F), lambda i: (0, 0)),
            pl.BlockSpec((F, H), lambda i: (0, 0)),
            pl.BlockSpec((1, H), lambda i: (0, 0)),
        ],
        out_specs=[
            pl.BlockSpec((_TILE, H), lambda i: (i, 0)),
            pl.BlockSpec(memory_space=pltpu.SMEM, block_shape=(1, 1),
                         index_map=lambda i: (0, 0)),
        ],
        out_shape=[
            jax.ShapeDtypeStruct((T, H), jnp.float32),
            jax.ShapeDtypeStruct((1, 1), jnp.float32),
        ],
        scratch_shapes=[pltpu.VMEM((8, _LANES), jnp.float32)],
    )(xs, router_weights, W1, b1r, W2, b2r)

    capacity = max(int(T * 1.25 * 2 / _NE), 4)
    return (out.reshape(B, S, H), loss[0, 0],
            jnp.asarray(capacity, dtype=jnp.int32))
